# baseline (speedup 1.0000x reference)
import functools

import jax
import jax.numpy as jnp
from jax import lax
from jax.experimental import pallas as pl
from jax.experimental.pallas import tpu as pltpu
from jax.experimental.pallas import tpu_sc as plsc

NN = 10000
FF = 128
HH = 64
EE = 320000
ETT = 4096

NC = 2
NS = 16
NW = NC * NS
CHUNK = 128
CPW = 80
SEG = 16
EPAD = NW * CPW * CHUNK
NP = 10112
RS = NP // NS
DEGW = 8
EPS = 1e-5

_mesh = plsc.VectorSubcoreMesh(core_axis_name="c", subcore_axis_name="s")
_sc_params = pltpu.CompilerParams(use_tc_tiling_on_sc=False)


def _sc_degree(ones, dst2, zeros):
    nb = 4
    @functools.partial(
        pl.kernel,
        out_type=(jax.ShapeDtypeStruct((NP, DEGW), jnp.float32),
                  jax.ShapeDtypeStruct((NP, DEGW), jnp.float32)),
        mesh=_mesh,
        compiler_params=_sc_params,
        scratch_types=[
            pltpu.VMEM((SEG, CHUNK), jnp.int32),
            pltpu.VMEM((CHUNK, DEGW), jnp.float32),
            pltpu.VMEM_SHARED((NP, DEGW), jnp.float32),
            [pltpu.SemaphoreType.DMA] * nb,
        ],
    )
    def k(ones_hbm, dst_hbm, zeros_hbm, out0, out1, dstv, buf, agg, ssems):
        cid = lax.axis_index("c")
        sid = lax.axis_index("s")
        wid = sid * NC + cid
        pltpu.sync_copy(ones_hbm, buf)
        pltpu.sync_copy(zeros_hbm.at[pl.ds(sid * RS, RS)],
                        agg.at[pl.ds(sid * RS, RS)])
        plsc.subcore_barrier()

        def seg_body(s, carry):
            pltpu.sync_copy(dst_hbm.at[pl.ds(wid * CPW + s * SEG, SEG)], dstv)
            for b in range(nb):
                pltpu.async_copy(buf, agg.at[dstv.at[b]], ssems[b], add=True)

            def body(i, c):
                for b in range(nb):
                    jj = i * nb + b
                    pltpu.make_async_copy(buf, agg.at[dstv.at[0]],
                                          ssems[b]).wait()

                    @pl.when(jj + nb < SEG)
                    def _():
                        pltpu.async_copy(buf, agg.at[dstv.at[jj + nb]],
                                         ssems[b], add=True)
                return c

            lax.fori_loop(0, SEG // nb, body, 0)
            return carry

        lax.fori_loop(0, CPW // SEG, seg_body, 0)
        plsc.subcore_barrier()

        @pl.when(cid == 0)
        def _():
            pltpu.sync_copy(agg.at[pl.ds(sid * RS, RS)],
                            out0.at[pl.ds(sid * RS, RS)])

        @pl.when(cid == 1)
        def _():
            pltpu.sync_copy(agg.at[pl.ds(sid * RS, RS)],
                            out1.at[pl.ds(sid * RS, RS)])

    return k(ones, dst2, zeros)


def _sc_scatter(table, src2, dst2, zeros, D, nb):
    @functools.partial(
        pl.kernel,
        out_type=(jax.ShapeDtypeStruct((NP, D), jnp.float32),
                  jax.ShapeDtypeStruct((NP, D), jnp.float32)),
        mesh=_mesh,
        compiler_params=_sc_params,
        scratch_types=[
            pltpu.VMEM((SEG, CHUNK), jnp.int32),
            pltpu.VMEM((SEG, CHUNK), jnp.int32),
            [pltpu.VMEM((CHUNK, D), jnp.float32)] * nb,
            pltpu.VMEM_SHARED((NP, D), jnp.float32),
            [pltpu.SemaphoreType.DMA] * nb,
            [pltpu.SemaphoreType.DMA] * nb,
        ],
    )
    def k(table_hbm, src_hbm, dst_hbm, zeros_hbm, out0, out1,
          srcv, dstv, bufs, agg, gsems, ssems):
        cid = lax.axis_index("c")
        sid = lax.axis_index("s")
        wid = sid * NC + cid
        pltpu.sync_copy(zeros_hbm.at[pl.ds(sid * RS, RS)],
                        agg.at[pl.ds(sid * RS, RS)])
        plsc.subcore_barrier()

        def seg_body(s, carry):
            base = wid * CPW + s * SEG
            pltpu.sync_copy(src_hbm.at[pl.ds(base, SEG)], srcv)
            pltpu.sync_copy(dst_hbm.at[pl.ds(base, SEG)], dstv)
            for b in range(nb):
                pltpu.async_copy(table_hbm.at[srcv.at[b]], bufs[b], gsems[b])

            def body(i, c):
                for b in range(nb):
                    pltpu.make_async_copy(table_hbm.at[srcv.at[0]],
                                          bufs[b], gsems[b]).wait()
                    jj = i * nb + b
                    pltpu.async_copy(bufs[b], agg.at[dstv.at[jj]],
                                     ssems[b], add=True)
                for b in range(nb):
                    pltpu.make_async_copy(bufs[b], agg.at[dstv.at[0]],
                                          ssems[b]).wait()
                    jj = i * nb + b

                    @pl.when(jj + nb < SEG)
                    def _():
                        pltpu.async_copy(table_hbm.at[srcv.at[jj + nb]],
                                         bufs[b], gsems[b])
                return c

            lax.fori_loop(0, SEG // nb, body, 0)
            return carry

        lax.fori_loop(0, CPW // SEG, seg_body, 0)
        plsc.subcore_barrier()

        @pl.when(cid == 0)
        def _():
            pltpu.sync_copy(agg.at[pl.ds(sid * RS, RS)],
                            out0.at[pl.ds(sid * RS, RS)])

        @pl.when(cid == 1)
        def _():
            pltpu.sync_copy(agg.at[pl.ds(sid * RS, RS)],
                            out1.at[pl.ds(sid * RS, RS)])

    return k(table, src2, dst2, zeros)


def _dinv_body(d0_ref, d1_ref, out_ref):
    d = d0_ref[:, 0:1] + d1_ref[:, 0:1] + 1.0
    out_ref[...] = lax.rsqrt(d)


def _dinv(deg0, deg1):
    return pl.pallas_call(
        _dinv_body,
        out_shape=jax.ShapeDtypeStruct((NP, 1), jnp.float32),
    )(deg0, deg1)


def _prescale_mm_body(x_ref, w_ref, dinv_ref, out_ref):
    h = jnp.dot(x_ref[...], w_ref[...], preferred_element_type=jnp.float32)
    out_ref[...] = dinv_ref[...] * h


def _prescale_mm(xpad, W, dinv, D):
    return pl.pallas_call(
        _prescale_mm_body,
        grid=(NS,),
        in_specs=[
            pl.BlockSpec((RS, xpad.shape[1]), lambda i: (i, 0)),
            pl.BlockSpec((xpad.shape[1], D), lambda i: (0, 0)),
            pl.BlockSpec((RS, 1), lambda i: (i, 0)),
        ],
        out_specs=pl.BlockSpec((RS, D), lambda i: (i, 0)),
        out_shape=jax.ShapeDtypeStruct((NP, D), jnp.float32),
    )(xpad, W, dinv)


_NB = 10
_RB = NN // _NB


def _combine_body(a0_ref, a1_ref, hp_ref, dinv_ref, b_ref, t_ref, s_ref):
    i = pl.program_id(0)
    t = dinv_ref[...] * (a0_ref[...] + a1_ref[...] + hp_ref[...]) + b_ref[...]
    t_ref[...] = t

    @pl.when(i == 0)
    def _():
        s_ref[...] = jnp.zeros_like(s_ref)

    s_ref[0:1, :] += jnp.sum(t, axis=0, keepdims=True)
    s_ref[1:2, :] += jnp.sum(t * t, axis=0, keepdims=True)


def _combine(agg0, agg1, hp, dinv, bias, D):
    return pl.pallas_call(
        _combine_body,
        grid=(_NB,),
        in_specs=[
            pl.BlockSpec((_RB, D), lambda i: (i, 0)),
            pl.BlockSpec((_RB, D), lambda i: (i, 0)),
            pl.BlockSpec((_RB, D), lambda i: (i, 0)),
            pl.BlockSpec((_RB, 1), lambda i: (i, 0)),
            pl.BlockSpec((1, D), lambda i: (0, 0)),
        ],
        out_specs=(pl.BlockSpec((_RB, D), lambda i: (i, 0)),
                   pl.BlockSpec((2, D), lambda i: (0, 0))),
        out_shape=(jax.ShapeDtypeStruct((NN, D), jnp.float32),
                   jax.ShapeDtypeStruct((2, D), jnp.float32)),
    )(agg0, agg1, hp, dinv, bias)


def _norm_from_stats(t, s_ref, gnp_ref, n_rows):
    m = s_ref[0:1, :] * (1.0 / n_rows)
    ex2 = s_ref[1:2, :] * (1.0 / n_rows)
    a = m * gnp_ref[2:3, :]
    var = ex2 - 2.0 * a * m + a * a
    o = t - a
    return jnp.maximum(
        gnp_ref[0:1, :] * o * lax.rsqrt(var + EPS) + gnp_ref[1:2, :], 0.0)


def _gn_mm_body(t_ref, s_ref, gnp_ref, w_ref, dinv_ref, out_ref):
    y = _norm_from_stats(t_ref[...], s_ref, gnp_ref, float(NN))
    h = jnp.dot(y, w_ref[...], preferred_element_type=jnp.float32)
    out_ref[...] = dinv_ref[...] * h


def _gn_mm(t, s, gnp, W, dinv, Din, Dout):
    return pl.pallas_call(
        _gn_mm_body,
        grid=(_NB,),
        in_specs=[
            pl.BlockSpec((_RB, Din), lambda i: (i, 0)),
            pl.BlockSpec((2, Din), lambda i: (0, 0)),
            pl.BlockSpec((3, Din), lambda i: (0, 0)),
            pl.BlockSpec((Din, Dout), lambda i: (0, 0)),
            pl.BlockSpec((_RB, 1), lambda i: (i, 0)),
        ],
        out_specs=pl.BlockSpec((_RB, Dout), lambda i: (i, 0)),
        out_shape=jax.ShapeDtypeStruct((NN, Dout), jnp.float32),
    )(t, s, gnp, W, dinv)


def _feature_body(srct_ref, dstt_ref, xt_ref, w5_ref, b5_ref, gnp5_ref, p_ref):
    cols = lax.broadcasted_iota(jnp.int32, (ETT, FF), 1)
    ss = (srct_ref[...] == cols).astype(jnp.float32)
    sd = (dstt_ref[...] == cols).astype(jnp.float32)
    cmat = lax.dot_general(sd, ss, (((0,), (0,)), ((), ())),
                           preferred_element_type=jnp.float32)
    deg = jnp.sum(cmat, axis=1) + 1.0
    dinv = lax.rsqrt(deg)
    ii = lax.broadcasted_iota(jnp.int32, (FF, FF), 0)
    jj = lax.broadcasted_iota(jnp.int32, (FF, FF), 1)
    amat = dinv[:, None] * cmat * dinv[None, :]
    amat = amat + jnp.where(ii == jj, (dinv * dinv)[None, :], 0.0)
    h3 = jnp.dot(xt_ref[...], w5_ref[...], preferred_element_type=jnp.float32)
    t3 = jnp.dot(amat, h3, preferred_element_type=jnp.float32) + b5_ref[...]
    m = jnp.mean(t3, axis=0, keepdims=True)
    o = t3 - m * gnp5_ref[2:3, :]
    v = jnp.mean(o * o, axis=0, keepdims=True)
    ht3 = jnp.maximum(
        gnp5_ref[0:1, :] * o * lax.rsqrt(v + EPS) + gnp5_ref[1:2, :], 0.0)
    p_ref[...] = jnp.dot(amat, ht3, preferred_element_type=jnp.float32)


def _feature_branch(srct, dstt, x_t, W5, b5, gnp5):
    return pl.pallas_call(
        _feature_body,
        out_shape=jax.ShapeDtypeStruct((FF, HH), jnp.float32),
    )(srct, dstt, x_t, W5, b5, gnp5)


def _final_body(t_ref, s_ref, gnp4_ref, p_ref, w8_ref, b8_ref, gnp8_ref,
                out_ref):
    y = _norm_from_stats(t_ref[...], s_ref, gnp4_ref, float(NN))
    ut = lax.dot_general(w8_ref[...], p_ref[...], (((1,), (1,)), ((), ())),
                         preferred_element_type=jnp.float32)
    ut = ut + b8_ref[...]
    mr = jnp.mean(ut, axis=1, keepdims=True)
    o = ut - mr * gnp8_ref[:, 2:3]
    vr = jnp.mean(o * o, axis=1, keepdims=True)
    z = jnp.maximum(
        gnp8_ref[:, 0:1] * o * lax.rsqrt(vr + EPS) + gnp8_ref[:, 1:2], 0.0)
    out_ref[...] = y + z


def _final(t2, s2, gnp4, P, W8t, b8t, gnp8t):
    return pl.pallas_call(
        _final_body,
        grid=(_NB,),
        in_specs=[
            pl.BlockSpec((_RB, FF), lambda i: (i, 0)),
            pl.BlockSpec((2, FF), lambda i: (0, 0)),
            pl.BlockSpec((3, FF), lambda i: (0, 0)),
            pl.BlockSpec((FF, HH), lambda i: (0, 0)),
            pl.BlockSpec((_RB, HH), lambda i: (i, 0)),
            pl.BlockSpec((_RB, 1), lambda i: (i, 0)),
            pl.BlockSpec((_RB, 3), lambda i: (i, 0)),
        ],
        out_specs=pl.BlockSpec((_RB, FF), lambda i: (i, 0)),
        out_shape=jax.ShapeDtypeStruct((NN, FF), jnp.float32),
    )(t2, s2, gnp4, P, W8t, b8t, gnp8t)


def kernel(data, x, adj, x_t, adj_t, clustering,
           W1, b1, W4, b4, W5, b5, W8, b8,
           gn1_w, gn1_b, gn1_ms, gn4_w, gn4_b, gn4_ms,
           gn5_w, gn5_b, gn5_ms, gn8_w, gn8_b, gn8_ms):
    pad = jnp.full((EPAD - EE,), NN, dtype=jnp.int32)
    src2 = jnp.concatenate([adj[:, 0], pad]).reshape(NW * CPW, CHUNK)
    dst2 = jnp.concatenate([adj[:, 1], pad]).reshape(NW * CPW, CHUNK)
    ones = jnp.ones((CHUNK, DEGW), jnp.float32)
    zeros_deg = jnp.zeros((NP, DEGW), jnp.float32)
    zeros64 = jnp.zeros((NP, HH), jnp.float32)
    zeros128 = jnp.zeros((NP, FF), jnp.float32)
    xpad = jnp.pad(x, ((0, NP - NN), (0, 0)))
    gnp1 = jnp.stack([gn1_w, gn1_b, gn1_ms])
    gnp4 = jnp.stack([gn4_w, gn4_b, gn4_ms])
    gnp5 = jnp.stack([gn5_w, gn5_b, gn5_ms])
    gnp8t = jnp.stack([gn8_w, gn8_b, gn8_ms], axis=1)
    srct = adj_t[:, 0:1]
    dstt = adj_t[:, 1:2]

    deg0, deg1 = _sc_degree(ones, dst2, zeros_deg)
    dinv = _dinv(deg0, deg1)

    h1p = _prescale_mm(xpad, W1, dinv, HH)
    a0, a1 = _sc_scatter(h1p, src2, dst2, zeros64, HH, 4)
    t1, s1 = _combine(a0, a1, h1p, dinv, b1.reshape(1, HH), HH)

    h2 = _gn_mm(t1, s1, gnp1, W4, dinv, HH, FF)
    h2p = jnp.pad(h2, ((0, NP - NN), (0, 0)))
    a20, a21 = _sc_scatter(h2p, src2, dst2, zeros128, FF, 2)
    t2, s2 = _combine(a20, a21, h2p, dinv, b4.reshape(1, FF), FF)

    P = _feature_branch(srct, dstt, x_t, W5, b5.reshape(1, HH), gnp5)

    return _final(t2, s2, gnp4, P, W8.T, b8.reshape(NN, 1), gnp8t)


# speedup vs baseline: 1.3465x; 1.3465x over previous
import functools

import jax
import jax.numpy as jnp
from jax import lax
from jax.experimental import pallas as pl
from jax.experimental.pallas import tpu as pltpu
from jax.experimental.pallas import tpu_sc as plsc

NN = 10000
FF = 128
HH = 64
EE = 320000
ETT = 4096

NC = 2
NS = 16
NW = NC * NS
CHUNK = 128
CPW = 80
SEG = 16
EPAD = NW * CPW * CHUNK
NP = 10112
RS = NP // NS
DEGW = 8
EPS = 1e-5

_mesh = plsc.VectorSubcoreMesh(core_axis_name="c", subcore_axis_name="s")
_sc_params = pltpu.CompilerParams(use_tc_tiling_on_sc=False)


def _sc_degree(ones, dst2, zeros):
    @functools.partial(
        pl.kernel,
        out_type=(jax.ShapeDtypeStruct((NP, DEGW), jnp.float32),
                  jax.ShapeDtypeStruct((NP, DEGW), jnp.float32)),
        mesh=_mesh,
        compiler_params=_sc_params,
        scratch_types=[
            pltpu.VMEM((SEG, CHUNK), jnp.int32),
            pltpu.VMEM((CHUNK, DEGW), jnp.float32),
            pltpu.VMEM_SHARED((NP, DEGW), jnp.float32),
        ],
    )
    def k(ones_hbm, dst_hbm, zeros_hbm, out0, out1, dstv, buf, agg):
        cid = lax.axis_index("c")
        sid = lax.axis_index("s")
        wid = sid * NC + cid
        pltpu.sync_copy(ones_hbm, buf)
        pltpu.sync_copy(zeros_hbm.at[pl.ds(sid * RS, RS)],
                        agg.at[pl.ds(sid * RS, RS)])
        plsc.subcore_barrier()

        def seg_body(s, carry):
            pltpu.sync_copy(dst_hbm.at[pl.ds(wid * CPW + s * SEG, SEG)], dstv)

            def body(j, c):
                pltpu.sync_copy(buf, agg.at[dstv.at[j]], add=True)
                return c

            lax.fori_loop(0, SEG, body, 0)
            return carry

        lax.fori_loop(0, CPW // SEG, seg_body, 0)
        plsc.subcore_barrier()

        @pl.when(cid == 0)
        def _():
            pltpu.sync_copy(agg.at[pl.ds(sid * RS, RS)],
                            out0.at[pl.ds(sid * RS, RS)])

        @pl.when(cid == 1)
        def _():
            pltpu.sync_copy(agg.at[pl.ds(sid * RS, RS)],
                            out1.at[pl.ds(sid * RS, RS)])

    return k(ones, dst2, zeros)


def _sc_scatter(table, src2, dst2, zeros, D, nb):
    @functools.partial(
        pl.kernel,
        out_type=(jax.ShapeDtypeStruct((NP, D), jnp.float32),
                  jax.ShapeDtypeStruct((NP, D), jnp.float32)),
        mesh=_mesh,
        compiler_params=_sc_params,
        scratch_types=[
            pltpu.VMEM((SEG, CHUNK), jnp.int32),
            pltpu.VMEM((SEG, CHUNK), jnp.int32),
            [pltpu.VMEM((CHUNK, D), jnp.float32)] * nb,
            pltpu.VMEM_SHARED((NP, D), jnp.float32),
            [pltpu.SemaphoreType.DMA] * nb,
            [pltpu.SemaphoreType.DMA] * nb,
        ],
    )
    def k(table_hbm, src_hbm, dst_hbm, zeros_hbm, out0, out1,
          srcv, dstv, bufs, agg, gsems, ssems):
        cid = lax.axis_index("c")
        sid = lax.axis_index("s")
        wid = sid * NC + cid
        pltpu.sync_copy(zeros_hbm.at[pl.ds(sid * RS, RS)],
                        agg.at[pl.ds(sid * RS, RS)])
        plsc.subcore_barrier()

        def seg_body(s, carry):
            base = wid * CPW + s * SEG
            pltpu.sync_copy(src_hbm.at[pl.ds(base, SEG)], srcv)
            pltpu.sync_copy(dst_hbm.at[pl.ds(base, SEG)], dstv)

            def body(j, c):
                pltpu.async_copy(table_hbm.at[srcv.at[j]], bufs[0],
                                 gsems[0]).wait()
                pltpu.sync_copy(bufs[0], agg.at[dstv.at[j]], add=True)
                return c

            lax.fori_loop(0, SEG, body, 0)
            return carry

        lax.fori_loop(0, CPW // SEG, seg_body, 0)
        plsc.subcore_barrier()

        @pl.when(cid == 0)
        def _():
            pltpu.sync_copy(agg.at[pl.ds(sid * RS, RS)],
                            out0.at[pl.ds(sid * RS, RS)])

        @pl.when(cid == 1)
        def _():
            pltpu.sync_copy(agg.at[pl.ds(sid * RS, RS)],
                            out1.at[pl.ds(sid * RS, RS)])

    return k(table, src2, dst2, zeros)


def _dinv_body(d0_ref, d1_ref, out_ref):
    d = d0_ref[:, 0:1] + d1_ref[:, 0:1] + 1.0
    out_ref[...] = lax.rsqrt(d)


def _dinv(deg0, deg1):
    return pl.pallas_call(
        _dinv_body,
        out_shape=jax.ShapeDtypeStruct((NP, 1), jnp.float32),
    )(deg0, deg1)


def _prescale_mm_body(x_ref, w_ref, dinv_ref, out_ref):
    h = jnp.dot(x_ref[...], w_ref[...], preferred_element_type=jnp.float32)
    out_ref[...] = dinv_ref[...] * h


def _prescale_mm(xpad, W, dinv, D):
    return pl.pallas_call(
        _prescale_mm_body,
        grid=(NS,),
        in_specs=[
            pl.BlockSpec((RS, xpad.shape[1]), lambda i: (i, 0)),
            pl.BlockSpec((xpad.shape[1], D), lambda i: (0, 0)),
            pl.BlockSpec((RS, 1), lambda i: (i, 0)),
        ],
        out_specs=pl.BlockSpec((RS, D), lambda i: (i, 0)),
        out_shape=jax.ShapeDtypeStruct((NP, D), jnp.float32),
    )(xpad, W, dinv)


_NB = 10
_RB = NN // _NB


def _combine_body(a0_ref, a1_ref, hp_ref, dinv_ref, b_ref, t_ref, s_ref):
    i = pl.program_id(0)
    t = dinv_ref[...] * (a0_ref[...] + a1_ref[...] + hp_ref[...]) + b_ref[...]
    t_ref[...] = t

    @pl.when(i == 0)
    def _():
        s_ref[...] = jnp.zeros_like(s_ref)

    s_ref[0:1, :] += jnp.sum(t, axis=0, keepdims=True)
    s_ref[1:2, :] += jnp.sum(t * t, axis=0, keepdims=True)


def _combine(agg0, agg1, hp, dinv, bias, D):
    return pl.pallas_call(
        _combine_body,
        grid=(_NB,),
        in_specs=[
            pl.BlockSpec((_RB, D), lambda i: (i, 0)),
            pl.BlockSpec((_RB, D), lambda i: (i, 0)),
            pl.BlockSpec((_RB, D), lambda i: (i, 0)),
            pl.BlockSpec((_RB, 1), lambda i: (i, 0)),
            pl.BlockSpec((1, D), lambda i: (0, 0)),
        ],
        out_specs=(pl.BlockSpec((_RB, D), lambda i: (i, 0)),
                   pl.BlockSpec((2, D), lambda i: (0, 0))),
        out_shape=(jax.ShapeDtypeStruct((NN, D), jnp.float32),
                   jax.ShapeDtypeStruct((2, D), jnp.float32)),
    )(agg0, agg1, hp, dinv, bias)


def _norm_from_stats(t, s_ref, gnp_ref, n_rows):
    m = s_ref[0:1, :] * (1.0 / n_rows)
    ex2 = s_ref[1:2, :] * (1.0 / n_rows)
    a = m * gnp_ref[2:3, :]
    var = ex2 - 2.0 * a * m + a * a
    o = t - a
    return jnp.maximum(
        gnp_ref[0:1, :] * o * lax.rsqrt(var + EPS) + gnp_ref[1:2, :], 0.0)


def _gn_scale_body(t_ref, s_ref, gnp_ref, dinv_ref, out_ref):
    y = _norm_from_stats(t_ref[...], s_ref, gnp_ref, float(NN))
    out_ref[...] = dinv_ref[...] * y


def _gn_scale(t, s, gnp, dinv, D):
    return pl.pallas_call(
        _gn_scale_body,
        grid=(_NB,),
        in_specs=[
            pl.BlockSpec((_RB, D), lambda i: (i, 0)),
            pl.BlockSpec((2, D), lambda i: (0, 0)),
            pl.BlockSpec((3, D), lambda i: (0, 0)),
            pl.BlockSpec((_RB, 1), lambda i: (i, 0)),
        ],
        out_specs=pl.BlockSpec((_RB, D), lambda i: (i, 0)),
        out_shape=jax.ShapeDtypeStruct((NN, D), jnp.float32),
    )(t, s, gnp, dinv)


def _combine_mm_body(a0_ref, a1_ref, up_ref, dinv_ref, w_ref, b_ref,
                     t_ref, s_ref):
    i = pl.program_id(0)
    u = a0_ref[...] + a1_ref[...] + up_ref[...]
    h = jnp.dot(u, w_ref[...], preferred_element_type=jnp.float32)
    t = dinv_ref[...] * h + b_ref[...]
    t_ref[...] = t

    @pl.when(i == 0)
    def _():
        s_ref[...] = jnp.zeros_like(s_ref)

    s_ref[0:1, :] += jnp.sum(t, axis=0, keepdims=True)
    s_ref[1:2, :] += jnp.sum(t * t, axis=0, keepdims=True)


def _combine_mm(agg0, agg1, up, dinv, W, bias, Din, Dout):
    return pl.pallas_call(
        _combine_mm_body,
        grid=(_NB,),
        in_specs=[
            pl.BlockSpec((_RB, Din), lambda i: (i, 0)),
            pl.BlockSpec((_RB, Din), lambda i: (i, 0)),
            pl.BlockSpec((_RB, Din), lambda i: (i, 0)),
            pl.BlockSpec((_RB, 1), lambda i: (i, 0)),
            pl.BlockSpec((Din, Dout), lambda i: (0, 0)),
            pl.BlockSpec((1, Dout), lambda i: (0, 0)),
        ],
        out_specs=(pl.BlockSpec((_RB, Dout), lambda i: (i, 0)),
                   pl.BlockSpec((2, Dout), lambda i: (0, 0))),
        out_shape=(jax.ShapeDtypeStruct((NN, Dout), jnp.float32),
                   jax.ShapeDtypeStruct((2, Dout), jnp.float32)),
    )(agg0, agg1, up, dinv, W, bias)


def _feature_body(srct_ref, dstt_ref, xt_ref, w5_ref, b5_ref, gnp5_ref, p_ref):
    cols = lax.broadcasted_iota(jnp.int32, (ETT, FF), 1)
    ss = (srct_ref[...] == cols).astype(jnp.float32)
    sd = (dstt_ref[...] == cols).astype(jnp.float32)
    cmat = lax.dot_general(sd, ss, (((0,), (0,)), ((), ())),
                           preferred_element_type=jnp.float32)
    deg = jnp.sum(cmat, axis=1) + 1.0
    dinv = lax.rsqrt(deg)
    ii = lax.broadcasted_iota(jnp.int32, (FF, FF), 0)
    jj = lax.broadcasted_iota(jnp.int32, (FF, FF), 1)
    amat = dinv[:, None] * cmat * dinv[None, :]
    amat = amat + jnp.where(ii == jj, (dinv * dinv)[None, :], 0.0)
    h3 = jnp.dot(xt_ref[...], w5_ref[...], preferred_element_type=jnp.float32)
    t3 = jnp.dot(amat, h3, preferred_element_type=jnp.float32) + b5_ref[...]
    m = jnp.mean(t3, axis=0, keepdims=True)
    o = t3 - m * gnp5_ref[2:3, :]
    v = jnp.mean(o * o, axis=0, keepdims=True)
    ht3 = jnp.maximum(
        gnp5_ref[0:1, :] * o * lax.rsqrt(v + EPS) + gnp5_ref[1:2, :], 0.0)
    p_ref[...] = jnp.dot(amat, ht3, preferred_element_type=jnp.float32)


def _feature_branch(srct, dstt, x_t, W5, b5, gnp5):
    return pl.pallas_call(
        _feature_body,
        out_shape=jax.ShapeDtypeStruct((FF, HH), jnp.float32),
    )(srct, dstt, x_t, W5, b5, gnp5)


def _final_body(t_ref, s_ref, gnp4_ref, p_ref, w8_ref, b8_ref, gnp8_ref,
                out_ref):
    y = _norm_from_stats(t_ref[...], s_ref, gnp4_ref, float(NN))
    ut = lax.dot_general(w8_ref[...], p_ref[...], (((1,), (1,)), ((), ())),
                         preferred_element_type=jnp.float32)
    ut = ut + b8_ref[...]
    mr = jnp.mean(ut, axis=1, keepdims=True)
    o = ut - mr * gnp8_ref[:, 2:3]
    vr = jnp.mean(o * o, axis=1, keepdims=True)
    z = jnp.maximum(
        gnp8_ref[:, 0:1] * o * lax.rsqrt(vr + EPS) + gnp8_ref[:, 1:2], 0.0)
    out_ref[...] = y + z


def _final(t2, s2, gnp4, P, W8t, b8t, gnp8t):
    return pl.pallas_call(
        _final_body,
        grid=(_NB,),
        in_specs=[
            pl.BlockSpec((_RB, FF), lambda i: (i, 0)),
            pl.BlockSpec((2, FF), lambda i: (0, 0)),
            pl.BlockSpec((3, FF), lambda i: (0, 0)),
            pl.BlockSpec((FF, HH), lambda i: (0, 0)),
            pl.BlockSpec((_RB, HH), lambda i: (i, 0)),
            pl.BlockSpec((_RB, 1), lambda i: (i, 0)),
            pl.BlockSpec((_RB, 3), lambda i: (i, 0)),
        ],
        out_specs=pl.BlockSpec((_RB, FF), lambda i: (i, 0)),
        out_shape=jax.ShapeDtypeStruct((NN, FF), jnp.float32),
    )(t2, s2, gnp4, P, W8t, b8t, gnp8t)


def kernel(data, x, adj, x_t, adj_t, clustering,
           W1, b1, W4, b4, W5, b5, W8, b8,
           gn1_w, gn1_b, gn1_ms, gn4_w, gn4_b, gn4_ms,
           gn5_w, gn5_b, gn5_ms, gn8_w, gn8_b, gn8_ms):
    pad = jnp.full((EPAD - EE,), NN, dtype=jnp.int32)
    src2 = jnp.concatenate([adj[:, 0], pad]).reshape(NW * CPW, CHUNK)
    dst2 = jnp.concatenate([adj[:, 1], pad]).reshape(NW * CPW, CHUNK)
    ones = jnp.ones((CHUNK, DEGW), jnp.float32)
    zeros_deg = jnp.zeros((NP, DEGW), jnp.float32)
    zeros64 = jnp.zeros((NP, HH), jnp.float32)
    xpad = jnp.pad(x, ((0, NP - NN), (0, 0)))
    gnp1 = jnp.stack([gn1_w, gn1_b, gn1_ms])
    gnp4 = jnp.stack([gn4_w, gn4_b, gn4_ms])
    gnp5 = jnp.stack([gn5_w, gn5_b, gn5_ms])
    gnp8t = jnp.stack([gn8_w, gn8_b, gn8_ms], axis=1)
    srct = adj_t[:, 0:1]
    dstt = adj_t[:, 1:2]

    deg0, deg1 = _sc_degree(ones, dst2, zeros_deg)
    dinv = _dinv(deg0, deg1)

    h1p = _prescale_mm(xpad, W1, dinv, HH)
    a0, a1 = _sc_scatter(h1p, src2, dst2, zeros64, HH, 1)
    t1, s1 = _combine(a0, a1, h1p, dinv, b1.reshape(1, HH), HH)

    u = _gn_scale(t1, s1, gnp1, dinv, HH)
    upad = jnp.pad(u, ((0, NP - NN), (0, 0)))
    a20, a21 = _sc_scatter(upad, src2, dst2, zeros64, HH, 1)
    t2, s2 = _combine_mm(a20, a21, upad, dinv, W4,
                         b4.reshape(1, FF), HH, FF)

    P = _feature_branch(srct, dstt, x_t, W5, b5.reshape(1, HH), gnp5)

    return _final(t2, s2, gnp4, P, W8.T, b8.reshape(NN, 1), gnp8t)


# speedup vs baseline: 1.4968x; 1.1117x over previous
import functools

import jax
import jax.numpy as jnp
from jax import lax
from jax.experimental import pallas as pl
from jax.experimental.pallas import tpu as pltpu
from jax.experimental.pallas import tpu_sc as plsc

NN = 10000
FF = 128
HH = 64
EE = 320000
ETT = 4096

NC = 2
NS = 16
NW = NC * NS
CHUNK = 128
CPW = 80
EPAD = NW * CPW * CHUNK
NP = 10112
RS = NP // NS
DEGW = 8
EPS = 1e-5

_mesh = plsc.VectorSubcoreMesh(core_axis_name="c", subcore_axis_name="s")
_sc_params = pltpu.CompilerParams(use_tc_tiling_on_sc=False)


def _sc_degree(ones, dst2, zeros):
    @functools.partial(
        pl.kernel,
        out_type=(jax.ShapeDtypeStruct((NP, DEGW), jnp.float32),
                  jax.ShapeDtypeStruct((NP, DEGW), jnp.float32)),
        mesh=_mesh,
        compiler_params=_sc_params,
        scratch_types=[
            pltpu.VMEM((CPW, CHUNK), jnp.int32),
            pltpu.VMEM((CHUNK, DEGW), jnp.float32),
            pltpu.VMEM_SHARED((NP, DEGW), jnp.float32),
        ],
    )
    def k(ones_hbm, dst_hbm, zeros_hbm, out0, out1, dstv, buf, agg):
        cid = lax.axis_index("c")
        sid = lax.axis_index("s")
        wid = sid * NC + cid
        pltpu.sync_copy(dst_hbm.at[pl.ds(wid * CPW, CPW)], dstv)
        pltpu.sync_copy(ones_hbm, buf)
        pltpu.sync_copy(zeros_hbm.at[pl.ds(sid * RS, RS)],
                        agg.at[pl.ds(sid * RS, RS)])
        plsc.subcore_barrier()

        def body(j, c):
            pltpu.sync_copy(buf, agg.at[dstv.at[j]], add=True)
            return c

        lax.fori_loop(0, CPW, body, 0)
        plsc.subcore_barrier()

        @pl.when(cid == 0)
        def _():
            pltpu.sync_copy(agg.at[pl.ds(sid * RS, RS)],
                            out0.at[pl.ds(sid * RS, RS)])

        @pl.when(cid == 1)
        def _():
            pltpu.sync_copy(agg.at[pl.ds(sid * RS, RS)],
                            out1.at[pl.ds(sid * RS, RS)])

    return k(ones, dst2, zeros)


def _sc_scatter(table, src2, dst2, zeros, D):
    @functools.partial(
        pl.kernel,
        out_type=(jax.ShapeDtypeStruct((NP, D), jnp.float32),
                  jax.ShapeDtypeStruct((NP, D), jnp.float32)),
        mesh=_mesh,
        compiler_params=_sc_params,
        scratch_types=[
            pltpu.VMEM((CPW, CHUNK), jnp.int32),
            pltpu.VMEM((CPW, CHUNK), jnp.int32),
            [pltpu.VMEM((CHUNK, D), jnp.float32)] * 2,
            pltpu.VMEM_SHARED((NP, D), jnp.float32),
            [pltpu.SemaphoreType.DMA] * 2,
        ],
    )
    def k(table_hbm, src_hbm, dst_hbm, zeros_hbm, out0, out1,
          srcv, dstv, bufs, agg, gsems):
        cid = lax.axis_index("c")
        sid = lax.axis_index("s")
        wid = sid * NC + cid
        pltpu.sync_copy(src_hbm.at[pl.ds(wid * CPW, CPW)], srcv)
        pltpu.sync_copy(dst_hbm.at[pl.ds(wid * CPW, CPW)], dstv)
        pltpu.sync_copy(zeros_hbm.at[pl.ds(sid * RS, RS)],
                        agg.at[pl.ds(sid * RS, RS)])
        plsc.subcore_barrier()

        pltpu.async_copy(table_hbm.at[srcv.at[0]], bufs[0], gsems[0])

        def body(i, c):
            for b in range(2):
                jj = i * 2 + b
                pltpu.make_async_copy(table_hbm.at[srcv.at[0]],
                                      bufs[b], gsems[b]).wait()

                @pl.when(jj + 1 < CPW)
                def _():
                    pltpu.async_copy(table_hbm.at[srcv.at[jj + 1]],
                                     bufs[1 - b], gsems[1 - b])
                pltpu.sync_copy(bufs[b], agg.at[dstv.at[jj]], add=True)
            return c

        lax.fori_loop(0, CPW // 2, body, 0)
        plsc.subcore_barrier()

        @pl.when(cid == 0)
        def _():
            pltpu.sync_copy(agg.at[pl.ds(sid * RS, RS)],
                            out0.at[pl.ds(sid * RS, RS)])

        @pl.when(cid == 1)
        def _():
            pltpu.sync_copy(agg.at[pl.ds(sid * RS, RS)],
                            out1.at[pl.ds(sid * RS, RS)])

    return k(table, src2, dst2, zeros)


def _dinv_body(d0_ref, d1_ref, out_ref):
    d = d0_ref[:, 0:1] + d1_ref[:, 0:1] + 1.0
    out_ref[...] = lax.rsqrt(d)


def _dinv(deg0, deg1):
    return pl.pallas_call(
        _dinv_body,
        out_shape=jax.ShapeDtypeStruct((NP, 1), jnp.float32),
    )(deg0, deg1)


def _prescale_mm_body(x_ref, w_ref, dinv_ref, out_ref):
    h = jnp.dot(x_ref[...], w_ref[...], preferred_element_type=jnp.float32)
    out_ref[...] = dinv_ref[...] * h


def _prescale_mm(xpad, W, dinv, D):
    return pl.pallas_call(
        _prescale_mm_body,
        grid=(NS,),
        in_specs=[
            pl.BlockSpec((RS, xpad.shape[1]), lambda i: (i, 0)),
            pl.BlockSpec((xpad.shape[1], D), lambda i: (0, 0)),
            pl.BlockSpec((RS, 1), lambda i: (i, 0)),
        ],
        out_specs=pl.BlockSpec((RS, D), lambda i: (i, 0)),
        out_shape=jax.ShapeDtypeStruct((NP, D), jnp.float32),
    )(xpad, W, dinv)


_NB = 10
_RB = NN // _NB


def _combine_body(a0_ref, a1_ref, hp_ref, dinv_ref, b_ref, t_ref, s_ref):
    i = pl.program_id(0)
    t = dinv_ref[...] * (a0_ref[...] + a1_ref[...] + hp_ref[...]) + b_ref[...]
    t_ref[...] = t

    @pl.when(i == 0)
    def _():
        s_ref[...] = jnp.zeros_like(s_ref)

    s_ref[0:1, :] += jnp.sum(t, axis=0, keepdims=True)
    s_ref[1:2, :] += jnp.sum(t * t, axis=0, keepdims=True)


def _combine(agg0, agg1, hp, dinv, bias, D):
    return pl.pallas_call(
        _combine_body,
        grid=(_NB,),
        in_specs=[
            pl.BlockSpec((_RB, D), lambda i: (i, 0)),
            pl.BlockSpec((_RB, D), lambda i: (i, 0)),
            pl.BlockSpec((_RB, D), lambda i: (i, 0)),
            pl.BlockSpec((_RB, 1), lambda i: (i, 0)),
            pl.BlockSpec((1, D), lambda i: (0, 0)),
        ],
        out_specs=(pl.BlockSpec((_RB, D), lambda i: (i, 0)),
                   pl.BlockSpec((2, D), lambda i: (0, 0))),
        out_shape=(jax.ShapeDtypeStruct((NN, D), jnp.float32),
                   jax.ShapeDtypeStruct((2, D), jnp.float32)),
    )(agg0, agg1, hp, dinv, bias)


def _norm_from_stats(t, s_ref, gnp_ref, n_rows):
    m = s_ref[0:1, :] * (1.0 / n_rows)
    ex2 = s_ref[1:2, :] * (1.0 / n_rows)
    a = m * gnp_ref[2:3, :]
    var = ex2 - 2.0 * a * m + a * a
    o = t - a
    return jnp.maximum(
        gnp_ref[0:1, :] * o * lax.rsqrt(var + EPS) + gnp_ref[1:2, :], 0.0)


def _gn_scale_body(t_ref, s_ref, gnp_ref, dinv_ref, out_ref):
    y = _norm_from_stats(t_ref[...], s_ref, gnp_ref, float(NN))
    out_ref[...] = dinv_ref[...] * y


def _gn_scale(t, s, gnp, dinv, D):
    return pl.pallas_call(
        _gn_scale_body,
        grid=(_NB,),
        in_specs=[
            pl.BlockSpec((_RB, D), lambda i: (i, 0)),
            pl.BlockSpec((2, D), lambda i: (0, 0)),
            pl.BlockSpec((3, D), lambda i: (0, 0)),
            pl.BlockSpec((_RB, 1), lambda i: (i, 0)),
        ],
        out_specs=pl.BlockSpec((_RB, D), lambda i: (i, 0)),
        out_shape=jax.ShapeDtypeStruct((NN, D), jnp.float32),
    )(t, s, gnp, dinv)


def _combine_mm_body(a0_ref, a1_ref, up_ref, dinv_ref, w_ref, b_ref,
                     t_ref, s_ref):
    i = pl.program_id(0)
    u = a0_ref[...] + a1_ref[...] + up_ref[...]
    h = jnp.dot(u, w_ref[...], preferred_element_type=jnp.float32)
    t = dinv_ref[...] * h + b_ref[...]
    t_ref[...] = t

    @pl.when(i == 0)
    def _():
        s_ref[...] = jnp.zeros_like(s_ref)

    s_ref[0:1, :] += jnp.sum(t, axis=0, keepdims=True)
    s_ref[1:2, :] += jnp.sum(t * t, axis=0, keepdims=True)


def _combine_mm(agg0, agg1, up, dinv, W, bias, Din, Dout):
    return pl.pallas_call(
        _combine_mm_body,
        grid=(_NB,),
        in_specs=[
            pl.BlockSpec((_RB, Din), lambda i: (i, 0)),
            pl.BlockSpec((_RB, Din), lambda i: (i, 0)),
            pl.BlockSpec((_RB, Din), lambda i: (i, 0)),
            pl.BlockSpec((_RB, 1), lambda i: (i, 0)),
            pl.BlockSpec((Din, Dout), lambda i: (0, 0)),
            pl.BlockSpec((1, Dout), lambda i: (0, 0)),
        ],
        out_specs=(pl.BlockSpec((_RB, Dout), lambda i: (i, 0)),
                   pl.BlockSpec((2, Dout), lambda i: (0, 0))),
        out_shape=(jax.ShapeDtypeStruct((NN, Dout), jnp.float32),
                   jax.ShapeDtypeStruct((2, Dout), jnp.float32)),
    )(agg0, agg1, up, dinv, W, bias)


def _feature_body(srct_ref, dstt_ref, xt_ref, w5_ref, b5_ref, gnp5_ref, p_ref):
    cols = lax.broadcasted_iota(jnp.int32, (ETT, FF), 1)
    ss = (srct_ref[...] == cols).astype(jnp.float32)
    sd = (dstt_ref[...] == cols).astype(jnp.float32)
    cmat = lax.dot_general(sd, ss, (((0,), (0,)), ((), ())),
                           preferred_element_type=jnp.float32)
    deg = jnp.sum(cmat, axis=1) + 1.0
    dinv = lax.rsqrt(deg)
    ii = lax.broadcasted_iota(jnp.int32, (FF, FF), 0)
    jj = lax.broadcasted_iota(jnp.int32, (FF, FF), 1)
    amat = dinv[:, None] * cmat * dinv[None, :]
    amat = amat + jnp.where(ii == jj, (dinv * dinv)[None, :], 0.0)
    h3 = jnp.dot(xt_ref[...], w5_ref[...], preferred_element_type=jnp.float32)
    t3 = jnp.dot(amat, h3, preferred_element_type=jnp.float32) + b5_ref[...]
    m = jnp.mean(t3, axis=0, keepdims=True)
    o = t3 - m * gnp5_ref[2:3, :]
    v = jnp.mean(o * o, axis=0, keepdims=True)
    ht3 = jnp.maximum(
        gnp5_ref[0:1, :] * o * lax.rsqrt(v + EPS) + gnp5_ref[1:2, :], 0.0)
    p_ref[...] = jnp.dot(amat, ht3, preferred_element_type=jnp.float32)


def _feature_branch(srct, dstt, x_t, W5, b5, gnp5):
    return pl.pallas_call(
        _feature_body,
        out_shape=jax.ShapeDtypeStruct((FF, HH), jnp.float32),
    )(srct, dstt, x_t, W5, b5, gnp5)


def _final_body(t_ref, s_ref, gnp4_ref, p_ref, w8_ref, b8_ref, gnp8_ref,
                out_ref):
    y = _norm_from_stats(t_ref[...], s_ref, gnp4_ref, float(NN))
    ut = lax.dot_general(w8_ref[...], p_ref[...], (((1,), (1,)), ((), ())),
                         preferred_element_type=jnp.float32)
    ut = ut + b8_ref[...]
    mr = jnp.mean(ut, axis=1, keepdims=True)
    o = ut - mr * gnp8_ref[:, 2:3]
    vr = jnp.mean(o * o, axis=1, keepdims=True)
    z = jnp.maximum(
        gnp8_ref[:, 0:1] * o * lax.rsqrt(vr + EPS) + gnp8_ref[:, 1:2], 0.0)
    out_ref[...] = y + z


def _final(t2, s2, gnp4, P, W8t, b8t, gnp8t):
    return pl.pallas_call(
        _final_body,
        grid=(_NB,),
        in_specs=[
            pl.BlockSpec((_RB, FF), lambda i: (i, 0)),
            pl.BlockSpec((2, FF), lambda i: (0, 0)),
            pl.BlockSpec((3, FF), lambda i: (0, 0)),
            pl.BlockSpec((FF, HH), lambda i: (0, 0)),
            pl.BlockSpec((_RB, HH), lambda i: (i, 0)),
            pl.BlockSpec((_RB, 1), lambda i: (i, 0)),
            pl.BlockSpec((_RB, 3), lambda i: (i, 0)),
        ],
        out_specs=pl.BlockSpec((_RB, FF), lambda i: (i, 0)),
        out_shape=jax.ShapeDtypeStruct((NN, FF), jnp.float32),
    )(t2, s2, gnp4, P, W8t, b8t, gnp8t)


def kernel(data, x, adj, x_t, adj_t, clustering,
           W1, b1, W4, b4, W5, b5, W8, b8,
           gn1_w, gn1_b, gn1_ms, gn4_w, gn4_b, gn4_ms,
           gn5_w, gn5_b, gn5_ms, gn8_w, gn8_b, gn8_ms):
    pad = jnp.full((EPAD - EE,), NN, dtype=jnp.int32)
    src2 = jnp.concatenate([adj[:, 0], pad]).reshape(NW * CPW, CHUNK)
    dst2 = jnp.concatenate([adj[:, 1], pad]).reshape(NW * CPW, CHUNK)
    ones = jnp.ones((CHUNK, DEGW), jnp.float32)
    zeros_deg = jnp.zeros((NP, DEGW), jnp.float32)
    zeros64 = jnp.zeros((NP, HH), jnp.float32)
    xpad = jnp.pad(x, ((0, NP - NN), (0, 0)))
    gnp1 = jnp.stack([gn1_w, gn1_b, gn1_ms])
    gnp4 = jnp.stack([gn4_w, gn4_b, gn4_ms])
    gnp5 = jnp.stack([gn5_w, gn5_b, gn5_ms])
    gnp8t = jnp.stack([gn8_w, gn8_b, gn8_ms], axis=1)
    srct = adj_t[:, 0:1]
    dstt = adj_t[:, 1:2]

    deg0, deg1 = _sc_degree(ones, dst2, zeros_deg)
    dinv = _dinv(deg0, deg1)

    h1p = _prescale_mm(xpad, W1, dinv, HH)
    a0, a1 = _sc_scatter(h1p, src2, dst2, zeros64, HH)
    t1, s1 = _combine(a0, a1, h1p, dinv, b1.reshape(1, HH), HH)

    u = _gn_scale(t1, s1, gnp1, dinv, HH)
    upad = jnp.pad(u, ((0, NP - NN), (0, 0)))
    a20, a21 = _sc_scatter(upad, src2, dst2, zeros64, HH)
    t2, s2 = _combine_mm(a20, a21, upad, dinv, W4,
                         b4.reshape(1, FF), HH, FF)

    P = _feature_branch(srct, dstt, x_t, W5, b5.reshape(1, HH), gnp5)

    return _final(t2, s2, gnp4, P, W8.T, b8.reshape(NN, 1), gnp8t)
